# baseline (speedup 1.0000x reference)
import functools

import jax
import jax.numpy as jnp
from jax import lax
from jax.experimental import pallas as pl
from jax.experimental.pallas import tpu as pltpu
from jax.experimental.pallas import tpu_sc as plsc

BATCH = 16384
VOCAB = 100000
EMB = 128
CTX = 6
OUT = 91

NUM_WORKERS = 32
NCH = 2
BCH = BATCH // NCH
BSPAN = BCH // NUM_WORKERS
CHUNK = 256
SUB = BSPAN // CHUNK


def _gather_body(off, table_hbm, idxT_hbm, out_hbm, idx_v, rows_a, rows_b, sem_a, sem_b):
    wid = lax.axis_index("s") * 2 + lax.axis_index("c")
    b0 = off + wid * BSPAN
    for t in range(CTX):
        pltpu.sync_copy(idxT_hbm.at[t, pl.ds(b0, BSPAN)],
                        idx_v.at[pl.ds(t * BSPAN, BSPAN)])
    bufs = (rows_a, rows_b)
    sems = (sem_a, sem_b)
    nsteps = CTX * SUB
    cps = [None, None]

    def src(step):
        return table_hbm.at[idx_v.at[pl.ds(step * CHUNK, CHUNK)]]

    def dst(step):
        t, s = divmod(step, SUB)
        return out_hbm.at[pl.ds(t * BCH + wid * BSPAN + s * CHUNK, CHUNK)]

    cps[0] = pltpu.async_copy(src(0), bufs[0], sems[0])
    for c in range(nsteps):
        nxt = (c + 1) % 2
        if c + 1 < nsteps:
            cps[nxt] = pltpu.async_copy(src(c + 1), bufs[nxt], sems[nxt])
        cps[c % 2].wait()
        pltpu.sync_copy(bufs[c % 2], dst(c))


def _make_gather(off):
    return pl.kernel(
        functools.partial(_gather_body, off),
        out_type=jax.ShapeDtypeStruct((CTX * BCH, EMB), jnp.float32),
        mesh=plsc.VectorSubcoreMesh(core_axis_name="c", subcore_axis_name="s"),
        scratch_types=[
            pltpu.VMEM((CTX * BSPAN,), jnp.int32),
            pltpu.VMEM((CHUNK, EMB), jnp.float32),
            pltpu.VMEM((CHUNK, EMB), jnp.float32),
            pltpu.SemaphoreType.DMA,
            pltpu.SemaphoreType.DMA,
        ],
    )


_gathers = [_make_gather(c * BCH) for c in range(NCH)]


def _mlp_body(x_ref, w1_ref, b1_ref, w2_ref, b2_ref, out_ref):
    acc = b1_ref[...].astype(jnp.float32)
    h = jnp.broadcast_to(acc, (x_ref.shape[1], EMB))
    for j in range(CTX):
        h = h + jax.lax.dot_general(
            x_ref[j], w1_ref[j], (((1,), (0,)), ((), ())),
            preferred_element_type=jnp.float32)
    h = jnp.maximum(h, 0.0)
    logits = jax.lax.dot_general(h, w2_ref[...], (((1,), (0,)), ((), ())),
                                 preferred_element_type=jnp.float32) + b2_ref[...]
    m = jnp.max(logits, axis=1, keepdims=True)
    s = logits - m
    lse = jnp.log(jnp.sum(jnp.exp(s), axis=1, keepdims=True))
    out_ref[...] = s - lse


BLOCK_B = 1024


def _mlp(x3, W1r, b1, W2, b2):
    grid = (BCH // BLOCK_B,)
    return pl.pallas_call(
        _mlp_body,
        grid=grid,
        in_specs=[
            pl.BlockSpec((CTX, BLOCK_B, EMB), lambda i: (0, i, 0)),
            pl.BlockSpec((CTX, EMB, EMB), lambda i: (0, 0, 0)),
            pl.BlockSpec((1, EMB), lambda i: (0, 0)),
            pl.BlockSpec((EMB, OUT), lambda i: (0, 0)),
            pl.BlockSpec((1, OUT), lambda i: (0, 0)),
        ],
        out_specs=pl.BlockSpec((BLOCK_B, OUT), lambda i: (i, 0)),
        out_shape=jax.ShapeDtypeStruct((BCH, OUT), jnp.float32),
    )(x3, W1r, b1, W2, b2)


@jax.jit
def kernel(inputs, emb, W1, b1, W2, b2):
    idxT = jnp.transpose(inputs)
    W1r = W1.reshape(CTX, EMB, EMB)
    b1r = b1.reshape(1, EMB)
    b2r = b2.reshape(1, OUT)
    outs = []
    for c in range(NCH):
        g = _gathers[c](emb, idxT)
        x3 = g.reshape(CTX, BCH, EMB)
        outs.append(_mlp(x3, W1r, b1r, W2, b2r))
    return jnp.concatenate(outs, axis=0)
